# baseline (speedup 1.0000x reference)
import functools

import jax
import jax.numpy as jnp
from jax import lax
from jax.experimental import pallas as pl
from jax.experimental.pallas import tpu as pltpu
from jax.experimental.pallas import tpu_sc as plsc

_NCOLS = 26
_NINT = 13
_VOCAB = 1_000_000
_BATCH = 16384

_NC, _NS, _L = 2, 16, 16
_NW = _NC * _NS
_ROWS_W = _BATCH // _NW
_PER_W = _ROWS_W * _NINT
_TOTAL = _BATCH * _NINT
_CHUNK = 128
_NCHUNK = _ROWS_W // _CHUNK

_W = 8192
_G = -(-_VOCAB // _W)
_RPAD = _G * _W


def _detile_body(in_ref, *out_refs):
    for j in range(_NINT):
        out_refs[j][...] = in_ref[j, :]


_detile = pl.pallas_call(
    _detile_body,
    grid=(_G,),
    in_specs=[pl.BlockSpec((_NINT, _W), lambda g: (0, g))],
    out_specs=[pl.BlockSpec((_W,), lambda g: (g,)) for _ in range(_NINT)],
    out_shape=[jax.ShapeDtypeStruct((_RPAD,), jnp.float32)
               for _ in range(_NINT)],
    compiler_params=pltpu.CompilerParams(
        dimension_semantics=("parallel",),
    ),
)

_mesh = plsc.VectorSubcoreMesh(core_axis_name="c", subcore_axis_name="s")


@functools.partial(
    pl.kernel,
    out_type=jax.ShapeDtypeStruct((_TOTAL,), jnp.float32),
    mesh=_mesh,
    scratch_types=[
        pltpu.VMEM((_PER_W,), jnp.float32),
        pltpu.VMEM((_PER_W,), jnp.int32),
        pltpu.VMEM((_PER_W,), jnp.float32),
        pltpu.SemaphoreType.DMA,
        pltpu.SemaphoreType.DMA,
    ],
)
def _sc_lookup(*refs):
    tables = refs[:_NINT]
    vals_hbm, out_hbm, v_vmem, idx_vmem, g_vmem, sem_io, sem_g = refs[_NINT:]
    wid = lax.axis_index("s") * _NC + lax.axis_index("c")
    rbase = wid * _ROWS_W

    @pl.loop(0, _NINT)
    def _(j):
        pltpu.async_copy(
            vals_hbm.at[pl.ds(j * _BATCH + rbase, _ROWS_W)],
            v_vmem.at[pl.ds(j * _ROWS_W, _ROWS_W)],
            sem_io,
        )

    pltpu.make_async_copy(vals_hbm.at[pl.ds(0, _PER_W)], v_vmem, sem_io).wait()

    @pl.loop(0, _PER_W, step=_L)
    def _(o):
        idx_vmem[pl.ds(o, _L)] = v_vmem[pl.ds(o, _L)].astype(jnp.int32)

    for j in range(_NINT):
        @pl.loop(0, _NCHUNK)
        def _(q, j=j):
            o = j * _ROWS_W + q * _CHUNK
            pltpu.async_copy(
                tables[j].at[idx_vmem.at[pl.ds(o, _CHUNK)]],
                g_vmem.at[pl.ds(o, _CHUNK)],
                sem_g,
            )

    pltpu.make_async_copy(vals_hbm.at[pl.ds(0, _PER_W)], g_vmem, sem_g).wait()

    @pl.loop(0, _NINT)
    def _(j):
        pltpu.async_copy(
            g_vmem.at[pl.ds(j * _ROWS_W, _ROWS_W)],
            out_hbm.at[pl.ds(j * _BATCH + rbase, _ROWS_W)],
            sem_io,
        )

    pltpu.make_async_copy(g_vmem, out_hbm.at[pl.ds(0, _PER_W)], sem_io).wait()


def kernel(inputs, lookup_tables):
    tables = _detile(lookup_tables)
    int_vals = inputs[:, 0::2].T.reshape(-1)
    looked_t = _sc_lookup(*tables, int_vals)
    looked = looked_t.reshape(_NINT, _BATCH).T
    num_vals = inputs[:, 1::2]
    num_vals = jnp.where(jnp.isnan(num_vals), 0.0, num_vals)
    return jnp.stack([looked, num_vals], axis=2).reshape(_BATCH, _NCOLS)


# speedup vs baseline: 1.5272x; 1.5272x over previous
import functools

import jax
import jax.numpy as jnp
from jax import lax
from jax.experimental import pallas as pl
from jax.experimental.pallas import tpu as pltpu
from jax.experimental.pallas import tpu_sc as plsc

_NCOLS = 26
_NINT = 13
_VOCAB = 1_000_000
_BATCH = 16384

_NC, _NS, _L = 2, 16, 16
_NW = _NC * _NS
_ROWS_W = _BATCH // _NW
_PER_W = _ROWS_W * _NINT
_TOTAL = _BATCH * _NINT
_CHUNK = 128
_NCHUNK = _ROWS_W // _CHUNK

_W = 32768
_G = -(-_VOCAB // _W)
_RPAD = _G * _W


def _detile_body(in_ref, *out_refs):
    for j in range(_NINT):
        out_refs[j][...] = in_ref[j, :]


_detile = pl.pallas_call(
    _detile_body,
    grid=(_G,),
    in_specs=[pl.BlockSpec((_NINT, _W), lambda g: (0, g))],
    out_specs=[pl.BlockSpec((_W,), lambda g: (g,)) for _ in range(_NINT)],
    out_shape=[jax.ShapeDtypeStruct((_RPAD,), jnp.float32)
               for _ in range(_NINT)],
    compiler_params=pltpu.CompilerParams(
        dimension_semantics=("parallel",),
    ),
)

_mesh = plsc.VectorSubcoreMesh(core_axis_name="c", subcore_axis_name="s")


@functools.partial(
    pl.kernel,
    out_type=jax.ShapeDtypeStruct((_TOTAL,), jnp.float32),
    mesh=_mesh,
    scratch_types=[
        pltpu.VMEM((_PER_W,), jnp.float32),
        pltpu.VMEM((_PER_W,), jnp.int32),
        pltpu.VMEM((_PER_W,), jnp.float32),
        pltpu.SemaphoreType.DMA,
        pltpu.SemaphoreType.DMA,
    ],
)
def _sc_lookup(*refs):
    tables = refs[:_NINT]
    vals_hbm, out_hbm, v_vmem, idx_vmem, g_vmem, sem_io, sem_g = refs[_NINT:]
    wid = lax.axis_index("s") * _NC + lax.axis_index("c")
    rbase = wid * _ROWS_W

    @pl.loop(0, _NINT)
    def _(j):
        pltpu.async_copy(
            vals_hbm.at[pl.ds(j * _BATCH + rbase, _ROWS_W)],
            v_vmem.at[pl.ds(j * _ROWS_W, _ROWS_W)],
            sem_io,
        )

    pltpu.make_async_copy(vals_hbm.at[pl.ds(0, _PER_W)], v_vmem, sem_io).wait()

    @pl.loop(0, _PER_W, step=_L)
    def _(o):
        idx_vmem[pl.ds(o, _L)] = v_vmem[pl.ds(o, _L)].astype(jnp.int32)

    for j in range(_NINT):
        @pl.loop(0, _NCHUNK)
        def _(q, j=j):
            o = j * _ROWS_W + q * _CHUNK
            pltpu.async_copy(
                tables[j].at[idx_vmem.at[pl.ds(o, _CHUNK)]],
                g_vmem.at[pl.ds(o, _CHUNK)],
                sem_g,
            )

    pltpu.make_async_copy(vals_hbm.at[pl.ds(0, _PER_W)], g_vmem, sem_g).wait()

    @pl.loop(0, _NINT)
    def _(j):
        pltpu.async_copy(
            g_vmem.at[pl.ds(j * _ROWS_W, _ROWS_W)],
            out_hbm.at[pl.ds(j * _BATCH + rbase, _ROWS_W)],
            sem_io,
        )

    pltpu.make_async_copy(g_vmem, out_hbm.at[pl.ds(0, _PER_W)], sem_io).wait()


def kernel(inputs, lookup_tables):
    tables = _detile(lookup_tables)
    int_vals = inputs[:, 0::2].T.reshape(-1)
    looked_t = _sc_lookup(*tables, int_vals)
    looked = looked_t.reshape(_NINT, _BATCH).T
    num_vals = inputs[:, 1::2]
    num_vals = jnp.where(jnp.isnan(num_vals), 0.0, num_vals)
    return jnp.stack([looked, num_vals], axis=2).reshape(_BATCH, _NCOLS)


# speedup vs baseline: 1.6191x; 1.0602x over previous
import functools

import jax
import jax.numpy as jnp
from jax import lax
from jax.experimental import pallas as pl
from jax.experimental.pallas import tpu as pltpu
from jax.experimental.pallas import tpu_sc as plsc

_NCOLS = 26
_NINT = 13
_VOCAB = 1_000_000
_BATCH = 16384

_NC, _NS, _L = 2, 16, 16
_NW = _NC * _NS
_ROWS_W = _BATCH // _NW
_PER_W = _ROWS_W * _NINT
_TOTAL = _BATCH * _NINT
_CHUNK = 128
_NCHUNK = _ROWS_W // _CHUNK

_W = 65536
_G = -(-_VOCAB // _W)
_RPAD = _G * _W


def _detile_body(in_ref, *out_refs):
    for j in range(_NINT):
        out_refs[j][...] = in_ref[j, :]


_detile = pl.pallas_call(
    _detile_body,
    grid=(_G,),
    in_specs=[pl.BlockSpec((_NINT, _W), lambda g: (0, g))],
    out_specs=[pl.BlockSpec((_W,), lambda g: (g,)) for _ in range(_NINT)],
    out_shape=[jax.ShapeDtypeStruct((_RPAD,), jnp.float32)
               for _ in range(_NINT)],
    compiler_params=pltpu.CompilerParams(
        dimension_semantics=("parallel",),
    ),
)

_mesh = plsc.VectorSubcoreMesh(core_axis_name="c", subcore_axis_name="s")


@functools.partial(
    pl.kernel,
    out_type=jax.ShapeDtypeStruct((_TOTAL,), jnp.float32),
    mesh=_mesh,
    scratch_types=[
        pltpu.VMEM((_PER_W,), jnp.float32),
        pltpu.VMEM((_PER_W,), jnp.int32),
        pltpu.VMEM((_PER_W,), jnp.float32),
        pltpu.SemaphoreType.DMA,
        pltpu.SemaphoreType.DMA,
    ],
)
def _sc_lookup(*refs):
    tables = refs[:_NINT]
    vals_hbm, out_hbm, v_vmem, idx_vmem, g_vmem, sem_io, sem_g = refs[_NINT:]
    wid = lax.axis_index("s") * _NC + lax.axis_index("c")
    rbase = wid * _ROWS_W

    @pl.loop(0, _NINT)
    def _(j):
        pltpu.async_copy(
            vals_hbm.at[pl.ds(j * _BATCH + rbase, _ROWS_W)],
            v_vmem.at[pl.ds(j * _ROWS_W, _ROWS_W)],
            sem_io,
        )

    pltpu.make_async_copy(vals_hbm.at[pl.ds(0, _PER_W)], v_vmem, sem_io).wait()

    @pl.loop(0, _PER_W, step=_L)
    def _(o):
        idx_vmem[pl.ds(o, _L)] = v_vmem[pl.ds(o, _L)].astype(jnp.int32)

    for j in range(_NINT):
        @pl.loop(0, _NCHUNK)
        def _(q, j=j):
            o = j * _ROWS_W + q * _CHUNK
            pltpu.async_copy(
                tables[j].at[idx_vmem.at[pl.ds(o, _CHUNK)]],
                g_vmem.at[pl.ds(o, _CHUNK)],
                sem_g,
            )

    pltpu.make_async_copy(vals_hbm.at[pl.ds(0, _PER_W)], g_vmem, sem_g).wait()

    @pl.loop(0, _NINT)
    def _(j):
        pltpu.async_copy(
            g_vmem.at[pl.ds(j * _ROWS_W, _ROWS_W)],
            out_hbm.at[pl.ds(j * _BATCH + rbase, _ROWS_W)],
            sem_io,
        )

    pltpu.make_async_copy(g_vmem, out_hbm.at[pl.ds(0, _PER_W)], sem_io).wait()


def kernel(inputs, lookup_tables):
    tables = _detile(lookup_tables)
    int_vals = inputs[:, 0::2].T.reshape(-1)
    looked_t = _sc_lookup(*tables, int_vals)
    looked = looked_t.reshape(_NINT, _BATCH).T
    num_vals = inputs[:, 1::2]
    num_vals = jnp.where(jnp.isnan(num_vals), 0.0, num_vals)
    return jnp.stack([looked, num_vals], axis=2).reshape(_BATCH, _NCOLS)


# speedup vs baseline: 1.6545x; 1.0218x over previous
import functools

import jax
import jax.numpy as jnp
from jax import lax
from jax.experimental import pallas as pl
from jax.experimental.pallas import tpu as pltpu
from jax.experimental.pallas import tpu_sc as plsc

_NCOLS = 26
_NINT = 13
_VOCAB = 1_000_000
_BATCH = 16384

_NC, _NS, _L = 2, 16, 16
_NW = _NC * _NS
_ROWS_W = _BATCH // _NW
_PER_W = _ROWS_W * _NINT
_TOTAL = _BATCH * _NINT
_CHUNK = 128
_NCHUNK = _ROWS_W // _CHUNK

_W = 131072
_G = -(-_VOCAB // _W)
_RPAD = _G * _W


def _detile_body(in_ref, *out_refs):
    for j in range(_NINT):
        out_refs[j][...] = in_ref[j, :]


_detile = pl.pallas_call(
    _detile_body,
    grid=(_G,),
    in_specs=[pl.BlockSpec((_NINT, _W), lambda g: (0, g))],
    out_specs=[pl.BlockSpec((_W,), lambda g: (g,)) for _ in range(_NINT)],
    out_shape=[jax.ShapeDtypeStruct((_RPAD,), jnp.float32)
               for _ in range(_NINT)],
    compiler_params=pltpu.CompilerParams(
        dimension_semantics=("parallel",),
    ),
)

_mesh = plsc.VectorSubcoreMesh(core_axis_name="c", subcore_axis_name="s")


@functools.partial(
    pl.kernel,
    out_type=jax.ShapeDtypeStruct((_TOTAL,), jnp.float32),
    mesh=_mesh,
    scratch_types=[
        pltpu.VMEM((_PER_W,), jnp.float32),
        pltpu.VMEM((_PER_W,), jnp.int32),
        pltpu.VMEM((_PER_W,), jnp.float32),
        pltpu.SemaphoreType.DMA,
        pltpu.SemaphoreType.DMA,
    ],
)
def _sc_lookup(*refs):
    tables = refs[:_NINT]
    vals_hbm, out_hbm, v_vmem, idx_vmem, g_vmem, sem_io, sem_g = refs[_NINT:]
    wid = lax.axis_index("s") * _NC + lax.axis_index("c")
    rbase = wid * _ROWS_W

    @pl.loop(0, _NINT)
    def _(j):
        pltpu.async_copy(
            vals_hbm.at[pl.ds(j * _BATCH + rbase, _ROWS_W)],
            v_vmem.at[pl.ds(j * _ROWS_W, _ROWS_W)],
            sem_io,
        )

    pltpu.make_async_copy(vals_hbm.at[pl.ds(0, _PER_W)], v_vmem, sem_io).wait()

    @pl.loop(0, _PER_W, step=_L)
    def _(o):
        idx_vmem[pl.ds(o, _L)] = v_vmem[pl.ds(o, _L)].astype(jnp.int32)

    for j in range(_NINT):
        @pl.loop(0, _NCHUNK)
        def _(q, j=j):
            o = j * _ROWS_W + q * _CHUNK
            pltpu.async_copy(
                tables[j].at[idx_vmem.at[pl.ds(o, _CHUNK)]],
                g_vmem.at[pl.ds(o, _CHUNK)],
                sem_g,
            )

    pltpu.make_async_copy(vals_hbm.at[pl.ds(0, _PER_W)], g_vmem, sem_g).wait()

    @pl.loop(0, _NINT)
    def _(j):
        pltpu.async_copy(
            g_vmem.at[pl.ds(j * _ROWS_W, _ROWS_W)],
            out_hbm.at[pl.ds(j * _BATCH + rbase, _ROWS_W)],
            sem_io,
        )

    pltpu.make_async_copy(g_vmem, out_hbm.at[pl.ds(0, _PER_W)], sem_io).wait()


def kernel(inputs, lookup_tables):
    tables = _detile(lookup_tables)
    int_vals = inputs[:, 0::2].T.reshape(-1)
    looked_t = _sc_lookup(*tables, int_vals)
    looked = looked_t.reshape(_NINT, _BATCH).T
    num_vals = inputs[:, 1::2]
    num_vals = jnp.where(jnp.isnan(num_vals), 0.0, num_vals)
    return jnp.stack([looked, num_vals], axis=2).reshape(_BATCH, _NCOLS)
